# baseline (speedup 1.0000x reference)
import functools

import jax
import jax.numpy as jnp
from jax.experimental import pallas as pl

_BJ = 256


def _attend(asrc_col, asrc5_col, adT_row, adj_blk, ones_col, feat):
    ms = jnp.max(asrc_col) + adT_row
    b = jnp.maximum(ms, 0.2 * ms)
    u = asrc_col + (adT_row - b)
    v = asrc5_col + (0.2 * adT_row - b)
    em = jnp.exp(jnp.maximum(u, v)) * adj_blk
    dcol = jax.lax.dot_general(
        em, ones_col, (((0,), (0,)), ((), ())),
        preferred_element_type=jnp.float32)
    oh = jax.lax.dot_general(
        em, feat, (((0,), (0,)), ((), ())),
        preferred_element_type=jnp.float32)
    return oh * (1.0 / (dcol + 1e-16))


def _layer1_kern(heads, ch, x_ref, xb_ref, adj_ref, W1_ref, As_ref, Ad_ref,
                 b1_ref, out_ref):
    W1 = W1_ref[:]
    hfull = jnp.dot(x_ref[:], W1, preferred_element_type=jnp.float32)
    asrc = jnp.dot(hfull, As_ref[:], preferred_element_type=jnp.float32)
    hblk = jnp.dot(xb_ref[:], W1, preferred_element_type=jnp.float32)
    adT = jax.lax.dot_general(
        Ad_ref[:], hblk, (((0,), (1,)), ((), ())),
        preferred_element_type=jnp.float32)
    adj_blk = adj_ref[:]
    asrc5 = 0.2 * asrc
    ones_col = jnp.ones_like(asrc[:, 0:1])
    parts = []
    for h in range(heads):
        parts.append(_attend(asrc[:, h:h + 1], asrc5[:, h:h + 1],
                             adT[h:h + 1, :], adj_blk, ones_col,
                             hfull[:, h * ch:(h + 1) * ch]))
    o = jnp.concatenate(parts, axis=1) + b1_ref[:]
    out_ref[:] = jnp.where(o > 0, o, jnp.exp(o) - 1.0)


def _layer2_kern(h1_ref, h1b_ref, adj_ref, W2_ref, as2_ref, ad2_ref, b2_ref,
                 out_ref):
    W2 = W2_ref[:]
    h2full = jnp.dot(h1_ref[:], W2, preferred_element_type=jnp.float32)
    h2blk = jnp.dot(h1b_ref[:], W2, preferred_element_type=jnp.float32)
    asrc = jax.lax.dot_general(
        h2full, as2_ref[:], (((1,), (1,)), ((), ())),
        preferred_element_type=jnp.float32)
    adT = jax.lax.dot_general(
        ad2_ref[:], h2blk, (((1,), (1,)), ((), ())),
        preferred_element_type=jnp.float32)
    ones_col = jnp.ones_like(asrc)
    out_ref[:] = _attend(asrc, 0.2 * asrc, adT, adj_ref[:], ones_col,
                         h2full) + b2_ref[:]


def kernel(x, adj, W1, att_src1, att_dst1, b1, W2, att_src2, att_dst2, b2):
    n, f_in = x.shape
    heads, ch = att_src1.shape
    nc = W2.shape[1]
    grid = (n // _BJ,)

    eye = jnp.eye(heads, dtype=jnp.float32)
    As_full = (eye[:, None, :] * att_src1[:, :, None]).reshape(heads * ch, heads)
    Ad_full = (eye[:, None, :] * att_dst1[:, :, None]).reshape(heads * ch, heads)

    full = lambda r, c: pl.BlockSpec((r, c), lambda j: (0, 0))
    colblk = lambda r: pl.BlockSpec((r, _BJ), lambda j: (0, j))
    rowblk = lambda c: pl.BlockSpec((_BJ, c), lambda j: (j, 0))

    h1 = pl.pallas_call(
        functools.partial(_layer1_kern, heads, ch),
        grid=grid,
        in_specs=[full(n, f_in), rowblk(f_in), colblk(n),
                  full(f_in, heads * ch), full(heads * ch, heads),
                  full(heads * ch, heads), full(1, heads * ch)],
        out_specs=rowblk(heads * ch),
        out_shape=jax.ShapeDtypeStruct((n, heads * ch), jnp.float32),
    )(x, x, adj, W1, As_full, Ad_full, b1.reshape(1, -1))

    out = pl.pallas_call(
        _layer2_kern,
        grid=grid,
        in_specs=[full(n, heads * ch), rowblk(heads * ch), colblk(n),
                  full(heads * ch, nc), full(1, nc), full(1, nc),
                  full(1, nc)],
        out_specs=rowblk(nc),
        out_shape=jax.ShapeDtypeStruct((n, nc), jnp.float32),
    )(h1, h1, adj, W2, att_src2, att_dst2, b2.reshape(1, -1))
    return out


# speedup vs baseline: 1.2190x; 1.2190x over previous
import functools

import jax
import jax.numpy as jnp
from jax.experimental import pallas as pl
from jax.experimental.pallas import tpu as pltpu

_BJ = 256


def _attend(asrc_col, adT_row, adj_blk, feat):
    s = asrc_col + adT_row
    t = jnp.maximum(s, 0.2 * s)
    ms = jnp.max(asrc_col) + adT_row
    b = jnp.maximum(ms, 0.2 * ms)
    em = jnp.exp(t - b) * adj_blk
    d = jnp.sum(em, axis=0, keepdims=True)
    p = em * (1.0 / (d + 1e-16))
    return jax.lax.dot_general(
        p, feat, (((0,), (0,)), ((), ())), preferred_element_type=jnp.float32
    )


def _fused_kern(grid_half, heads, ch,
                x_ref, xb_ref, adj_ref, W1_ref, As_ref, Ad_ref, b1_ref,
                W2_ref, as2_ref, ad2_ref, b2_ref, out_ref, h1_scr):
    pid = pl.program_id(0)
    adj_blk = adj_ref[:]

    @pl.when(pid < grid_half)
    def _layer1():
        W1 = W1_ref[:]
        hfull = jnp.dot(x_ref[:], W1, preferred_element_type=jnp.float32)
        asrc = jnp.dot(hfull, As_ref[:], preferred_element_type=jnp.float32)
        hblk = jnp.dot(xb_ref[:], W1, preferred_element_type=jnp.float32)
        adT = jax.lax.dot_general(
            Ad_ref[:], hblk, (((0,), (1,)), ((), ())),
            preferred_element_type=jnp.float32)
        parts = []
        for h in range(heads):
            parts.append(_attend(asrc[:, h:h + 1], adT[h:h + 1, :], adj_blk,
                                 hfull[:, h * ch:(h + 1) * ch]))
        o = jnp.concatenate(parts, axis=1) + b1_ref[:]
        o = jnp.where(o > 0, o, jnp.exp(o) - 1.0)
        h1_scr[pl.ds(pid * _BJ, _BJ), :] = o

    @pl.when(pid >= grid_half)
    def _layer2():
        W2 = W2_ref[:]
        h1full = h1_scr[:]
        h1blk = h1_scr[pl.ds((pid - grid_half) * _BJ, _BJ), :]
        h2full = jnp.dot(h1full, W2, preferred_element_type=jnp.float32)
        h2blk = jnp.dot(h1blk, W2, preferred_element_type=jnp.float32)
        asrc = jax.lax.dot_general(
            h2full, as2_ref[:], (((1,), (1,)), ((), ())),
            preferred_element_type=jnp.float32)
        adT = jax.lax.dot_general(
            ad2_ref[:], h2blk, (((1,), (1,)), ((), ())),
            preferred_element_type=jnp.float32)
        out_ref[:] = _attend(asrc, adT, adj_blk, h2full) + b2_ref[:]


def kernel(x, adj, W1, att_src1, att_dst1, b1, W2, att_src2, att_dst2, b2):
    n, f_in = x.shape
    heads, ch = att_src1.shape
    nc = W2.shape[1]
    g = n // _BJ

    eye = jnp.eye(heads, dtype=jnp.float32)
    As_full = (eye[:, None, :] * att_src1[:, :, None]).reshape(heads * ch, heads)
    Ad_full = (eye[:, None, :] * att_dst1[:, :, None]).reshape(heads * ch, heads)

    full = lambda r, c: pl.BlockSpec((r, c), lambda j: (0, 0))

    out = pl.pallas_call(
        functools.partial(_fused_kern, g, heads, ch),
        grid=(2 * g,),
        in_specs=[
            full(n, f_in),
            pl.BlockSpec((_BJ, f_in), lambda j: (j % g, 0)),
            pl.BlockSpec((n, _BJ), lambda j: (0, j % g)),
            full(f_in, heads * ch), full(heads * ch, heads),
            full(heads * ch, heads), full(1, heads * ch),
            full(heads * ch, nc), full(1, nc), full(1, nc), full(1, nc),
        ],
        out_specs=pl.BlockSpec((_BJ, nc),
                               lambda j: (jnp.where(j < g, 0, j - g), 0)),
        out_shape=jax.ShapeDtypeStruct((n, nc), jnp.float32),
        scratch_shapes=[pltpu.VMEM((n, heads * ch), jnp.float32)],
    )(x, x, adj, W1, As_full, Ad_full, b1.reshape(1, -1),
      W2, att_src2, att_dst2, b2.reshape(1, -1))
    return out


# speedup vs baseline: 1.2980x; 1.0648x over previous
import functools

import jax
import jax.numpy as jnp
from jax.experimental import pallas as pl
from jax.experimental.pallas import tpu as pltpu

_BJ = 256


def _masked_exp(asrc_col, asrc5_col, adT_row, b_row, adj_blk):
    u = asrc_col + (adT_row - b_row)
    v = asrc5_col + (0.2 * adT_row - b_row)
    return jnp.exp(jnp.maximum(u, v)) * adj_blk


def _layer1(heads, ch, x_ref, xb_ref, adj_blk, W1_ref, As_ref, Ad_ref,
            b1_ref):
    W1 = W1_ref[:]
    hfull = jnp.dot(x_ref[:], W1, preferred_element_type=jnp.float32)
    asrc = jnp.dot(hfull, As_ref[:], preferred_element_type=jnp.float32)
    hblk = jnp.dot(xb_ref[:], W1, preferred_element_type=jnp.float32)
    adT = jax.lax.dot_general(
        Ad_ref[:], hblk, (((0,), (1,)), ((), ())),
        preferred_element_type=jnp.float32)
    asrc5 = 0.2 * asrc
    ms = jnp.max(asrc, axis=0, keepdims=True)
    parts, dens = [], []
    for h in range(heads):
        adT_h = adT[h:h + 1, :]
        mrow = ms[0:1, h:h + 1] + adT_h
        b_row = jnp.maximum(mrow, 0.2 * mrow)
        em = _masked_exp(asrc[:, h:h + 1], asrc5[:, h:h + 1], adT_h,
                         b_row, adj_blk)
        dens.append(jnp.sum(em, axis=0, keepdims=True))
        parts.append(jax.lax.dot_general(
            em, hfull[:, h * ch:(h + 1) * ch], (((0,), (0,)), ((), ())),
            preferred_element_type=jnp.float32))
    d_t = jnp.transpose(jnp.concatenate(dens, axis=0), (1, 0))
    r = 1.0 / (d_t + 1e-16)
    li = jax.lax.broadcasted_iota(jnp.int32, (heads, heads * ch), 1) // ch
    si = jax.lax.broadcasted_iota(jnp.int32, (heads, heads * ch), 0)
    rep = (li == si).astype(jnp.float32)
    scale = jnp.dot(r, rep, preferred_element_type=jnp.float32)
    o = jnp.concatenate(parts, axis=1) * scale + b1_ref[:]
    return jnp.where(o > 0, o, jnp.exp(o) - 1.0)


def _layer2(h1full, h1blk, adj_blk, W2_ref, as2_ref, ad2_ref, b2_ref):
    W2 = W2_ref[:]
    h2full = jnp.dot(h1full, W2, preferred_element_type=jnp.float32)
    h2blk = jnp.dot(h1blk, W2, preferred_element_type=jnp.float32)
    asrc = jax.lax.dot_general(
        h2full, as2_ref[:], (((1,), (1,)), ((), ())),
        preferred_element_type=jnp.float32)
    adT = jax.lax.dot_general(
        ad2_ref[:], h2blk, (((1,), (1,)), ((), ())),
        preferred_element_type=jnp.float32)
    mrow = jnp.max(asrc) + adT
    b_row = jnp.maximum(mrow, 0.2 * mrow)
    em = _masked_exp(asrc, 0.2 * asrc, adT, b_row, adj_blk)
    d = jnp.sum(em, axis=0, keepdims=True)
    oh = jax.lax.dot_general(
        em, h2full, (((0,), (0,)), ((), ())),
        preferred_element_type=jnp.float32)
    d_col = jnp.transpose(d, (1, 0))
    return oh * (1.0 / (d_col + 1e-16)) + b2_ref[:]


def _fused_kern(grid_half, heads, ch,
                x_ref, xb_ref, adj_ref, W1_ref, As_ref, Ad_ref, b1_ref,
                W2_ref, as2_ref, ad2_ref, b2_ref, out_ref, h1_scr):
    pid = pl.program_id(0)
    adj_blk = adj_ref[:]

    @pl.when(pid < grid_half)
    def _():
        h1_scr[pl.ds(pid * _BJ, _BJ), :] = _layer1(
            heads, ch, x_ref, xb_ref, adj_blk, W1_ref, As_ref, Ad_ref, b1_ref)

    @pl.when(pid >= grid_half)
    def _():
        h1blk = h1_scr[pl.ds((pid - grid_half) * _BJ, _BJ), :]
        out_ref[:] = _layer2(h1_scr[:], h1blk, adj_blk, W2_ref, as2_ref,
                             ad2_ref, b2_ref)


def kernel(x, adj, W1, att_src1, att_dst1, b1, W2, att_src2, att_dst2, b2):
    n, f_in = x.shape
    heads, ch = att_src1.shape
    nc = W2.shape[1]
    g = n // _BJ

    eye = jnp.eye(heads, dtype=jnp.float32)
    As_full = (eye[:, None, :] * att_src1[:, :, None]).reshape(heads * ch, heads)
    Ad_full = (eye[:, None, :] * att_dst1[:, :, None]).reshape(heads * ch, heads)

    full = lambda r, c: pl.BlockSpec((r, c), lambda j: (0, 0))

    out = pl.pallas_call(
        functools.partial(_fused_kern, g, heads, ch),
        grid=(2 * g,),
        in_specs=[
            full(n, f_in),
            pl.BlockSpec((_BJ, f_in), lambda j: (j % g, 0)),
            pl.BlockSpec((n, _BJ), lambda j: (0, j % g)),
            full(f_in, heads * ch), full(heads * ch, heads),
            full(heads * ch, heads), full(1, heads * ch),
            full(heads * ch, nc), full(1, nc), full(1, nc), full(1, nc),
        ],
        out_specs=pl.BlockSpec((_BJ, nc),
                               lambda j: (jnp.where(j < g, 0, j - g), 0)),
        out_shape=jax.ShapeDtypeStruct((n, nc), jnp.float32),
        scratch_shapes=[pltpu.VMEM((n, heads * ch), jnp.float32)],
    )(x, x, adj, W1, As_full, Ad_full, b1.reshape(1, -1),
      W2, att_src2, att_dst2, b2.reshape(1, -1))
    return out


# speedup vs baseline: 1.3072x; 1.0071x over previous
import functools

import jax
import jax.numpy as jnp
from jax.experimental import pallas as pl
from jax.experimental.pallas import tpu as pltpu

_BJ = 256


def _masked_exp(lhs, u_sel, v_sel, ones_sel, adT_row, b_row, adj_blk):
    k = u_sel.shape[0]
    adTb = jnp.broadcast_to(adT_row - b_row, (k, adT_row.shape[1]))
    adTb2 = jnp.broadcast_to(0.2 * adT_row - b_row, (k, adT_row.shape[1]))
    ru = u_sel + jnp.where(ones_sel, adTb, 0.0)
    rv = v_sel + jnp.where(ones_sel, adTb2, 0.0)
    u = jnp.dot(lhs, ru, preferred_element_type=jnp.float32)
    v = jnp.dot(lhs, rv, preferred_element_type=jnp.float32)
    return jnp.exp(jnp.maximum(u, v)) * adj_blk


def _layer1(heads, ch, x_ref, xb_ref, adj_blk, W1_ref, As_ref, Ad_ref,
            b1_ref):
    W1 = W1_ref[:]
    hfull = jnp.dot(x_ref[:], W1, preferred_element_type=jnp.float32)
    asrc = jnp.dot(hfull, As_ref[:], preferred_element_type=jnp.float32)
    hblk = jnp.dot(xb_ref[:], W1, preferred_element_type=jnp.float32)
    adT = jax.lax.dot_general(
        Ad_ref[:], hblk, (((0,), (1,)), ((), ())),
        preferred_element_type=jnp.float32)
    asrc5 = 0.2 * asrc
    ms = jnp.max(asrc, axis=0, keepdims=True)
    n = adj_blk.shape[0]
    k = 2 * heads + 1
    lhs = jnp.concatenate(
        [asrc, asrc5, jnp.ones((n, 1), jnp.float32)], axis=1)
    rowidx = jax.lax.broadcasted_iota(jnp.int32, (k, _BJ), 0)
    ones_sel = rowidx == 2 * heads
    parts, dens = [], []
    for h in range(heads):
        adT_h = adT[h:h + 1, :]
        mrow = ms[0:1, h:h + 1] + adT_h
        b_row = jnp.maximum(mrow, 0.2 * mrow)
        em = _masked_exp(lhs, (rowidx == h).astype(jnp.float32),
                         (rowidx == heads + h).astype(jnp.float32),
                         ones_sel, adT_h, b_row, adj_blk)
        dens.append(jnp.sum(em, axis=0, keepdims=True))
        parts.append(jax.lax.dot_general(
            em, hfull[:, h * ch:(h + 1) * ch], (((0,), (0,)), ((), ())),
            preferred_element_type=jnp.float32))
    d_t = jnp.transpose(jnp.concatenate(dens, axis=0), (1, 0))
    r = 1.0 / (d_t + 1e-16)
    li = jax.lax.broadcasted_iota(jnp.int32, (heads, heads * ch), 1) // ch
    si = jax.lax.broadcasted_iota(jnp.int32, (heads, heads * ch), 0)
    rep = (li == si).astype(jnp.float32)
    scale = jnp.dot(r, rep, preferred_element_type=jnp.float32)
    o = jnp.concatenate(parts, axis=1) * scale + b1_ref[:]
    return jnp.where(o > 0, o, jnp.exp(o) - 1.0)


def _layer2(h1full, h1blk, adj_blk, W2_ref, as2_ref, ad2_ref, b2_ref):
    W2 = W2_ref[:]
    h2full = jnp.dot(h1full, W2, preferred_element_type=jnp.float32)
    h2blk = jnp.dot(h1blk, W2, preferred_element_type=jnp.float32)
    asrc = jax.lax.dot_general(
        h2full, as2_ref[:], (((1,), (1,)), ((), ())),
        preferred_element_type=jnp.float32)
    adT = jax.lax.dot_general(
        ad2_ref[:], h2blk, (((1,), (1,)), ((), ())),
        preferred_element_type=jnp.float32)
    mrow = jnp.max(asrc) + adT
    b_row = jnp.maximum(mrow, 0.2 * mrow)
    n = adj_blk.shape[0]
    lhs = jnp.concatenate(
        [asrc, 0.2 * asrc, jnp.ones((n, 1), jnp.float32)], axis=1)
    rowidx = jax.lax.broadcasted_iota(jnp.int32, (3, _BJ), 0)
    em = _masked_exp(lhs, (rowidx == 0).astype(jnp.float32),
                     (rowidx == 1).astype(jnp.float32), rowidx == 2,
                     adT, b_row, adj_blk)
    d = jnp.sum(em, axis=0, keepdims=True)
    oh = jax.lax.dot_general(
        em, h2full, (((0,), (0,)), ((), ())),
        preferred_element_type=jnp.float32)
    d_col = jnp.transpose(d, (1, 0))
    return oh * (1.0 / (d_col + 1e-16)) + b2_ref[:]


def _fused_kern(grid_half, heads, ch,
                x_ref, xb_ref, adj_ref, W1_ref, As_ref, Ad_ref, b1_ref,
                W2_ref, as2_ref, ad2_ref, b2_ref, out_ref, h1_scr):
    pid = pl.program_id(0)
    adj_blk = adj_ref[:]

    @pl.when(pid < grid_half)
    def _():
        h1_scr[pl.ds(pid * _BJ, _BJ), :] = _layer1(
            heads, ch, x_ref, xb_ref, adj_blk, W1_ref, As_ref, Ad_ref, b1_ref)

    @pl.when(pid >= grid_half)
    def _():
        h1blk = h1_scr[pl.ds((pid - grid_half) * _BJ, _BJ), :]
        out_ref[:] = _layer2(h1_scr[:], h1blk, adj_blk, W2_ref, as2_ref,
                             ad2_ref, b2_ref)


def kernel(x, adj, W1, att_src1, att_dst1, b1, W2, att_src2, att_dst2, b2):
    n, f_in = x.shape
    heads, ch = att_src1.shape
    nc = W2.shape[1]
    g = n // _BJ

    eye = jnp.eye(heads, dtype=jnp.float32)
    As_full = (eye[:, None, :] * att_src1[:, :, None]).reshape(heads * ch, heads)
    Ad_full = (eye[:, None, :] * att_dst1[:, :, None]).reshape(heads * ch, heads)

    full = lambda r, c: pl.BlockSpec((r, c), lambda j: (0, 0))

    out = pl.pallas_call(
        functools.partial(_fused_kern, g, heads, ch),
        grid=(2 * g,),
        in_specs=[
            full(n, f_in),
            pl.BlockSpec((_BJ, f_in), lambda j: (j % g, 0)),
            pl.BlockSpec((n, _BJ), lambda j: (0, j % g)),
            full(f_in, heads * ch), full(heads * ch, heads),
            full(heads * ch, heads), full(1, heads * ch),
            full(heads * ch, nc), full(1, nc), full(1, nc), full(1, nc),
        ],
        out_specs=pl.BlockSpec((_BJ, nc),
                               lambda j: (jnp.where(j < g, 0, j - g), 0)),
        out_shape=jax.ShapeDtypeStruct((n, nc), jnp.float32),
        scratch_shapes=[pltpu.VMEM((n, heads * ch), jnp.float32)],
    )(x, x, adj, W1, As_full, Ad_full, b1.reshape(1, -1),
      W2, att_src2, att_dst2, b2.reshape(1, -1))
    return out


# speedup vs baseline: 1.4673x; 1.1225x over previous
import functools

import jax
import jax.numpy as jnp
from jax.experimental import pallas as pl
from jax.experimental.pallas import tpu as pltpu

_BJ = 512


def _masked_exp(lhs, u_sel, v_sel, ones_sel, adT_row, b_row, adj_blk):
    k = u_sel.shape[0]
    adTb = jnp.broadcast_to(adT_row - b_row, (k, adT_row.shape[1]))
    adTb2 = jnp.broadcast_to(0.2 * adT_row - b_row, (k, adT_row.shape[1]))
    ru = u_sel + jnp.where(ones_sel, adTb, 0.0)
    rv = v_sel + jnp.where(ones_sel, adTb2, 0.0)
    u = jnp.dot(lhs, ru, preferred_element_type=jnp.float32)
    v = jnp.dot(lhs, rv, preferred_element_type=jnp.float32)
    return jnp.exp(jnp.maximum(u, v)) * adj_blk


def _layer1(heads, ch, x_ref, xb_ref, adj_blk, W1_ref, As_ref, Ad_ref,
            b1_ref):
    W1 = W1_ref[:]
    hfull = jnp.dot(x_ref[:], W1, preferred_element_type=jnp.float32)
    asrc = jnp.dot(hfull, As_ref[:], preferred_element_type=jnp.float32)
    hblk = jnp.dot(xb_ref[:], W1, preferred_element_type=jnp.float32)
    adT = jax.lax.dot_general(
        Ad_ref[:], hblk, (((0,), (1,)), ((), ())),
        preferred_element_type=jnp.float32)
    asrc5 = 0.2 * asrc
    ms = jnp.max(asrc, axis=0, keepdims=True)
    n = adj_blk.shape[0]
    k = 2 * heads + 1
    lhs = jnp.concatenate(
        [asrc, asrc5, jnp.ones((n, 1), jnp.float32)], axis=1)
    rowidx = jax.lax.broadcasted_iota(jnp.int32, (k, _BJ), 0)
    ones_sel = rowidx == 2 * heads
    parts, dens = [], []
    for h in range(heads):
        adT_h = adT[h:h + 1, :]
        mrow = ms[0:1, h:h + 1] + adT_h
        b_row = jnp.maximum(mrow, 0.2 * mrow)
        em = _masked_exp(lhs, (rowidx == h).astype(jnp.float32),
                         (rowidx == heads + h).astype(jnp.float32),
                         ones_sel, adT_h, b_row, adj_blk)
        dens.append(jnp.sum(em, axis=0, keepdims=True))
        parts.append(jax.lax.dot_general(
            em, hfull[:, h * ch:(h + 1) * ch], (((0,), (0,)), ((), ())),
            preferred_element_type=jnp.float32))
    d_t = jnp.transpose(jnp.concatenate(dens, axis=0), (1, 0))
    r = 1.0 / (d_t + 1e-16)
    li = jax.lax.broadcasted_iota(jnp.int32, (heads, heads * ch), 1) // ch
    si = jax.lax.broadcasted_iota(jnp.int32, (heads, heads * ch), 0)
    rep = (li == si).astype(jnp.float32)
    scale = jnp.dot(r, rep, preferred_element_type=jnp.float32)
    o = jnp.concatenate(parts, axis=1) * scale + b1_ref[:]
    return jnp.where(o > 0, o, jnp.exp(o) - 1.0)


def _layer2(h1full, h1blk, adj_blk, W2_ref, as2_ref, ad2_ref, b2_ref):
    W2 = W2_ref[:]
    h2full = jnp.dot(h1full, W2, preferred_element_type=jnp.float32)
    h2blk = jnp.dot(h1blk, W2, preferred_element_type=jnp.float32)
    asrc = jax.lax.dot_general(
        h2full, as2_ref[:], (((1,), (1,)), ((), ())),
        preferred_element_type=jnp.float32)
    adT = jax.lax.dot_general(
        ad2_ref[:], h2blk, (((1,), (1,)), ((), ())),
        preferred_element_type=jnp.float32)
    mrow = jnp.max(asrc) + adT
    b_row = jnp.maximum(mrow, 0.2 * mrow)
    n = adj_blk.shape[0]
    lhs = jnp.concatenate(
        [asrc, 0.2 * asrc, jnp.ones((n, 1), jnp.float32)], axis=1)
    rowidx = jax.lax.broadcasted_iota(jnp.int32, (3, _BJ), 0)
    em = _masked_exp(lhs, (rowidx == 0).astype(jnp.float32),
                     (rowidx == 1).astype(jnp.float32), rowidx == 2,
                     adT, b_row, adj_blk)
    d = jnp.sum(em, axis=0, keepdims=True)
    oh = jax.lax.dot_general(
        em, h2full, (((0,), (0,)), ((), ())),
        preferred_element_type=jnp.float32)
    d_col = jnp.transpose(d, (1, 0))
    return oh * (1.0 / (d_col + 1e-16)) + b2_ref[:]


def _fused_kern(grid_half, heads, ch,
                x_ref, xb_ref, adj_ref, W1_ref, As_ref, Ad_ref, b1_ref,
                W2_ref, as2_ref, ad2_ref, b2_ref, out_ref, h1_scr):
    pid = pl.program_id(0)
    adj_blk = adj_ref[:]

    @pl.when(pid < grid_half)
    def _():
        h1_scr[pl.ds(pid * _BJ, _BJ), :] = _layer1(
            heads, ch, x_ref, xb_ref, adj_blk, W1_ref, As_ref, Ad_ref, b1_ref)

    @pl.when(pid >= grid_half)
    def _():
        h1blk = h1_scr[pl.ds((pid - grid_half) * _BJ, _BJ), :]
        out_ref[:] = _layer2(h1_scr[:], h1blk, adj_blk, W2_ref, as2_ref,
                             ad2_ref, b2_ref)


def kernel(x, adj, W1, att_src1, att_dst1, b1, W2, att_src2, att_dst2, b2):
    n, f_in = x.shape
    heads, ch = att_src1.shape
    nc = W2.shape[1]
    g = n // _BJ

    eye = jnp.eye(heads, dtype=jnp.float32)
    As_full = (eye[:, None, :] * att_src1[:, :, None]).reshape(heads * ch, heads)
    Ad_full = (eye[:, None, :] * att_dst1[:, :, None]).reshape(heads * ch, heads)

    full = lambda r, c: pl.BlockSpec((r, c), lambda j: (0, 0))

    out = pl.pallas_call(
        functools.partial(_fused_kern, g, heads, ch),
        grid=(2 * g,),
        in_specs=[
            full(n, f_in),
            pl.BlockSpec((_BJ, f_in), lambda j: (j % g, 0)),
            pl.BlockSpec((n, _BJ), lambda j: (0, j % g)),
            full(f_in, heads * ch), full(heads * ch, heads),
            full(heads * ch, heads), full(1, heads * ch),
            full(heads * ch, nc), full(1, nc), full(1, nc), full(1, nc),
        ],
        out_specs=pl.BlockSpec((_BJ, nc),
                               lambda j: (jnp.where(j < g, 0, j - g), 0)),
        out_shape=jax.ShapeDtypeStruct((n, nc), jnp.float32),
        scratch_shapes=[pltpu.VMEM((n, heads * ch), jnp.float32)],
    )(x, x, adj, W1, As_full, Ad_full, b1.reshape(1, -1),
      W2, att_src2, att_dst2, b2.reshape(1, -1))
    return out


# speedup vs baseline: 1.5221x; 1.0373x over previous
import jax
import jax.numpy as jnp
from jax.experimental import pallas as pl


def _masked_exp(lhs, u_sel, v_sel, ones_sel, adT_row, b_row, adj_blk):
    k, bj = u_sel.shape[0], adT_row.shape[1]
    adTb = jnp.broadcast_to(adT_row - b_row, (k, bj))
    adTb2 = jnp.broadcast_to(0.2 * adT_row - b_row, (k, bj))
    ru = u_sel + jnp.where(ones_sel, adTb, 0.0)
    rv = v_sel + jnp.where(ones_sel, adTb2, 0.0)
    u = jnp.dot(lhs, ru, preferred_element_type=jnp.float32)
    v = jnp.dot(lhs, rv, preferred_element_type=jnp.float32)
    return jnp.exp(jnp.maximum(u, v)) * adj_blk


def _layer1(heads, ch, x, adj, W1, As, Ad, b1):
    n = x.shape[0]
    hfull = jnp.dot(x, W1, preferred_element_type=jnp.float32)
    asrc = jnp.dot(hfull, As, preferred_element_type=jnp.float32)
    adT = jax.lax.dot_general(
        Ad, hfull, (((0,), (1,)), ((), ())),
        preferred_element_type=jnp.float32)
    asrc5 = 0.2 * asrc
    ms = jnp.max(asrc, axis=0, keepdims=True)
    k = 2 * heads + 1
    lhs = jnp.concatenate(
        [asrc, asrc5, jnp.ones((n, 1), jnp.float32)], axis=1)
    rowidx = jax.lax.broadcasted_iota(jnp.int32, (k, n), 0)
    ones_sel = rowidx == 2 * heads
    parts, dens = [], []
    for h in range(heads):
        adT_h = adT[h:h + 1, :]
        mrow = ms[0:1, h:h + 1] + adT_h
        b_row = jnp.maximum(mrow, 0.2 * mrow)
        em = _masked_exp(lhs, (rowidx == h).astype(jnp.float32),
                         (rowidx == heads + h).astype(jnp.float32),
                         ones_sel, adT_h, b_row, adj)
        dens.append(jnp.sum(em, axis=0, keepdims=True))
        parts.append(jax.lax.dot_general(
            em, hfull[:, h * ch:(h + 1) * ch], (((0,), (0,)), ((), ())),
            preferred_element_type=jnp.float32))
    d_t = jnp.transpose(jnp.concatenate(dens, axis=0), (1, 0))
    r = 1.0 / (d_t + 1e-16)
    li = jax.lax.broadcasted_iota(jnp.int32, (heads, heads * ch), 1) // ch
    si = jax.lax.broadcasted_iota(jnp.int32, (heads, heads * ch), 0)
    rep = (li == si).astype(jnp.float32)
    scale = jnp.dot(r, rep, preferred_element_type=jnp.float32)
    o = jnp.concatenate(parts, axis=1) * scale + b1
    return jnp.where(o > 0, o, jnp.exp(o) - 1.0)


def _layer2(h1, adj, W2, as2, ad2, b2):
    n = h1.shape[0]
    h2full = jnp.dot(h1, W2, preferred_element_type=jnp.float32)
    asrc = jax.lax.dot_general(
        h2full, as2, (((1,), (1,)), ((), ())),
        preferred_element_type=jnp.float32)
    adT = jax.lax.dot_general(
        ad2, h2full, (((1,), (1,)), ((), ())),
        preferred_element_type=jnp.float32)
    mrow = jnp.max(asrc) + adT
    b_row = jnp.maximum(mrow, 0.2 * mrow)
    lhs = jnp.concatenate(
        [asrc, 0.2 * asrc, jnp.ones((n, 1), jnp.float32)], axis=1)
    rowidx = jax.lax.broadcasted_iota(jnp.int32, (3, n), 0)
    em = _masked_exp(lhs, (rowidx == 0).astype(jnp.float32),
                     (rowidx == 1).astype(jnp.float32), rowidx == 2,
                     adT, b_row, adj)
    d = jnp.sum(em, axis=0, keepdims=True)
    oh = jax.lax.dot_general(
        em, h2full, (((0,), (0,)), ((), ())),
        preferred_element_type=jnp.float32)
    d_col = jnp.transpose(d, (1, 0))
    return oh * (1.0 / (d_col + 1e-16)) + b2


def _fused_kern(heads, ch, x_ref, adj_ref, W1_ref, As_ref, Ad_ref, b1_ref,
                W2_ref, as2_ref, ad2_ref, b2_ref, out_ref):
    adj = adj_ref[:]
    h1 = _layer1(heads, ch, x_ref[:], adj, W1_ref[:], As_ref[:], Ad_ref[:],
                 b1_ref[:])
    out_ref[:] = _layer2(h1, adj, W2_ref[:], as2_ref[:], ad2_ref[:],
                         b2_ref[:])


import functools


def kernel(x, adj, W1, att_src1, att_dst1, b1, W2, att_src2, att_dst2, b2):
    n, f_in = x.shape
    heads, ch = att_src1.shape
    nc = W2.shape[1]

    eye = jnp.eye(heads, dtype=jnp.float32)
    As_full = (eye[:, None, :] * att_src1[:, :, None]).reshape(heads * ch, heads)
    Ad_full = (eye[:, None, :] * att_dst1[:, :, None]).reshape(heads * ch, heads)

    out = pl.pallas_call(
        functools.partial(_fused_kern, heads, ch),
        out_shape=jax.ShapeDtypeStruct((n, nc), jnp.float32),
    )(x, adj, W1, As_full, Ad_full, b1.reshape(1, -1),
      W2, att_src2, att_dst2, b2.reshape(1, -1))
    return out


# speedup vs baseline: 1.5307x; 1.0057x over previous
import jax
import jax.numpy as jnp
from jax.experimental import pallas as pl


def _masked_exp(lhs, u_sel, v_sel, ones_sel, adT_row, b_row, adj_blk):
    k, bj = u_sel.shape[0], adT_row.shape[1]
    adTb = jnp.broadcast_to(adT_row - b_row, (k, bj))
    adTb2 = jnp.broadcast_to(0.2 * adT_row - b_row, (k, bj))
    ru = u_sel + jnp.where(ones_sel, adTb, 0.0)
    rv = v_sel + jnp.where(ones_sel, adTb2, 0.0)
    u = jnp.dot(lhs, ru, preferred_element_type=jnp.float32)
    v = jnp.dot(lhs, rv, preferred_element_type=jnp.float32)
    return jnp.exp(jnp.maximum(u, v)) * adj_blk


def _layer1(heads, ch, x, adj, W1, As, Ad, b1):
    n = x.shape[0]
    hfull = jnp.dot(x, W1, preferred_element_type=jnp.float32)
    asrc = jnp.dot(hfull, As, preferred_element_type=jnp.float32)
    adT = jax.lax.dot_general(
        Ad, hfull, (((0,), (1,)), ((), ())),
        preferred_element_type=jnp.float32)
    asrc5 = 0.2 * asrc
    ms = jnp.max(asrc, axis=0, keepdims=True)
    k = 2 * heads + 1
    ones_col = jnp.ones((n, 1), jnp.float32)
    lhs = jnp.concatenate([asrc, asrc5, ones_col], axis=1)
    rowidx = jax.lax.broadcasted_iota(jnp.int32, (k, n), 0)
    ones_sel = rowidx == 2 * heads
    aug = []
    for h in range(heads):
        aug += [hfull[:, h * ch:(h + 1) * ch], ones_col]
    aug = jnp.concatenate(aug, axis=1)
    parts = []
    for h in range(heads):
        adT_h = adT[h:h + 1, :]
        mrow = ms[0:1, h:h + 1] + adT_h
        b_row = jnp.maximum(mrow, 0.2 * mrow)
        em = _masked_exp(lhs, (rowidx == h).astype(jnp.float32),
                         (rowidx == heads + h).astype(jnp.float32),
                         ones_sel, adT_h, b_row, adj)
        ohd = jax.lax.dot_general(
            em, aug[:, h * (ch + 1):(h + 1) * (ch + 1)],
            (((0,), (0,)), ((), ())), preferred_element_type=jnp.float32)
        parts.append(ohd[:, :ch] * (1.0 / (ohd[:, ch:ch + 1] + 1e-16)))
    o = jnp.concatenate(parts, axis=1) + b1
    return jnp.where(o > 0, o, jnp.exp(o) - 1.0)


def _layer2(h1, adj, W2, as2, ad2, b2):
    n = h1.shape[0]
    h2full = jnp.dot(h1, W2, preferred_element_type=jnp.float32)
    asrc = jax.lax.dot_general(
        h2full, as2, (((1,), (1,)), ((), ())),
        preferred_element_type=jnp.float32)
    adT = jax.lax.dot_general(
        ad2, h2full, (((1,), (1,)), ((), ())),
        preferred_element_type=jnp.float32)
    mrow = jnp.max(asrc) + adT
    b_row = jnp.maximum(mrow, 0.2 * mrow)
    ones_col = jnp.ones((n, 1), jnp.float32)
    lhs = jnp.concatenate([asrc, 0.2 * asrc, ones_col], axis=1)
    rowidx = jax.lax.broadcasted_iota(jnp.int32, (3, n), 0)
    em = _masked_exp(lhs, (rowidx == 0).astype(jnp.float32),
                     (rowidx == 1).astype(jnp.float32), rowidx == 2,
                     adT, b_row, adj)
    nc = h2full.shape[1]
    aug = jnp.concatenate([h2full, ones_col], axis=1)
    ohd = jax.lax.dot_general(
        em, aug, (((0,), (0,)), ((), ())),
        preferred_element_type=jnp.float32)
    return ohd[:, :nc] * (1.0 / (ohd[:, nc:nc + 1] + 1e-16)) + b2


def _fused_kern(heads, ch, x_ref, adj_ref, W1_ref, As_ref, Ad_ref, b1_ref,
                W2_ref, as2_ref, ad2_ref, b2_ref, out_ref):
    adj = adj_ref[:]
    h1 = _layer1(heads, ch, x_ref[:], adj, W1_ref[:], As_ref[:], Ad_ref[:],
                 b1_ref[:])
    out_ref[:] = _layer2(h1, adj, W2_ref[:], as2_ref[:], ad2_ref[:],
                         b2_ref[:])


import functools


def kernel(x, adj, W1, att_src1, att_dst1, b1, W2, att_src2, att_dst2, b2):
    n, f_in = x.shape
    heads, ch = att_src1.shape
    nc = W2.shape[1]

    eye = jnp.eye(heads, dtype=jnp.float32)
    As_full = (eye[:, None, :] * att_src1[:, :, None]).reshape(heads * ch, heads)
    Ad_full = (eye[:, None, :] * att_dst1[:, :, None]).reshape(heads * ch, heads)

    out = pl.pallas_call(
        functools.partial(_fused_kern, heads, ch),
        out_shape=jax.ShapeDtypeStruct((n, nc), jnp.float32),
    )(x, adj, W1, As_full, Ad_full, b1.reshape(1, -1),
      W2, att_src2, att_dst2, b2.reshape(1, -1))
    return out
